# Initial kernel scaffold; baseline (speedup 1.0000x reference)
import jax
import jax.numpy as jnp
from jax.experimental import pallas as pl


def kernel(x, W1, b1, W2, b2, fW1, fb1, fW2, fb2, beta1, beta2, edge_index):
    raise NotImplementedError("write your pallas kernel here")


# speedup vs baseline: 30.3042x; 30.3042x over previous
import functools

import jax
import jax.numpy as jnp
from jax import lax
from jax.experimental import pallas as pl
from jax.experimental.pallas import tpu as pltpu
from jax.experimental.pallas import tpu_sc as plsc

N = 10000
IN_C = 128
HID_C = 256
OUT_C = 128
E = 320000

NC = 2
NS = 16
NW = NC * NS
WIN = 128
STEPS = 80
EPT = WIN * STEPS
E_PAD = EPT * NW
N_PAD = 10240
DEG_W = 16
NBUF = 2


@functools.cache
def _make_sc_degree():
    mesh = plsc.VectorSubcoreMesh(
        core_axis_name="core", subcore_axis_name="subcore")
    return pl.kernel(
        _sc_degree_body,
        out_type=jax.ShapeDtypeStruct((NC, N_PAD, IN_C), jnp.float32),
        mesh=mesh,
        scratch_types=[
            pltpu.VMEM_SHARED((N_PAD, IN_C), jnp.float32),
            pltpu.VMEM((STEPS, WIN), jnp.int32),
            pltpu.VMEM((WIN, IN_C), jnp.float32),
        ],
    )


def _sc_degree_body(dst_hbm, z_hbm, ones_hbm, out_hbm, acc, didx, ones):
    cid = lax.axis_index("core")
    sid = lax.axis_index("subcore")
    wid = cid * NS + sid
    rpt = N_PAD // NS
    r0 = sid * rpt
    pltpu.sync_copy(z_hbm.at[pl.ds(r0, rpt)], acc.at[pl.ds(r0, rpt)])
    pltpu.sync_copy(dst_hbm.at[wid], didx)
    pltpu.sync_copy(ones_hbm, ones)
    plsc.subcore_barrier()

    @pl.loop(0, STEPS)
    def _(t):
        pltpu.sync_copy(ones, acc.at[didx.at[t]], add=True)

    plsc.subcore_barrier()
    pltpu.sync_copy(acc.at[pl.ds(r0, rpt)], out_hbm.at[cid, pl.ds(r0, rpt)])


NSLOT = 4


@functools.cache
def _make_sc_aggregate():
    mesh = plsc.VectorSubcoreMesh(
        core_axis_name="core", subcore_axis_name="subcore")
    return pl.kernel(
        _sc_aggregate_body,
        out_type=jax.ShapeDtypeStruct((NC, N_PAD, IN_C), jnp.float32),
        mesh=mesh,
        scratch_types=[
            pltpu.VMEM_SHARED((N_PAD, IN_C), jnp.float32),
            pltpu.VMEM((NSLOT, 2, WIN), jnp.int32),
            pltpu.VMEM((NBUF, WIN, IN_C), jnp.float32),
            pltpu.SemaphoreType.DMA((NSLOT,)),
            pltpu.SemaphoreType.DMA((NBUF,)),
        ],
    )


def _sc_aggregate_body(ed_hbm, g_hbm, z_hbm, out_hbm,
                       acc, ebuf, rows, isem, gsem):
    cid = lax.axis_index("core")
    sid = lax.axis_index("subcore")
    wid = cid * NS + sid
    rpt = N_PAD // NS
    r0 = sid * rpt
    pltpu.sync_copy(z_hbm.at[pl.ds(r0, rpt)], acc.at[pl.ds(r0, rpt)])
    for k in range(NSLOT):
        pltpu.async_copy(ed_hbm.at[wid, k], ebuf.at[k], isem.at[k])
    for b in range(NBUF):
        pltpu.make_async_copy(ed_hbm.at[wid, b], ebuf.at[b], isem.at[b]).wait()
        pltpu.async_copy(g_hbm.at[ebuf.at[b, 0]], rows.at[b], gsem.at[b])
    plsc.subcore_barrier()

    @pl.loop(0, STEPS, step=NBUF)
    def _(t0):
        for b in range(NBUF):
            t = t0 + b
            slot = lax.rem(t, NSLOT)
            pltpu.make_async_copy(
                g_hbm.at[ebuf.at[slot, 0]], rows.at[b], gsem.at[b]).wait()
            pltpu.sync_copy(rows.at[b], acc.at[ebuf.at[slot, 1]], add=True)
            nt_idx = t + NSLOT

            @pl.when(nt_idx < STEPS)
            def _():
                pltpu.async_copy(
                    ed_hbm.at[wid, nt_idx], ebuf.at[slot], isem.at[slot])

            nt_g = t + NBUF

            @pl.when(nt_g < STEPS)
            def _():
                slot2 = lax.rem(nt_g, NSLOT)
                pltpu.make_async_copy(
                    ed_hbm.at[wid, nt_g], ebuf.at[slot2],
                    isem.at[slot2]).wait()
                pltpu.async_copy(
                    g_hbm.at[ebuf.at[slot2, 0]], rows.at[b], gsem.at[b])

    plsc.subcore_barrier()
    pltpu.sync_copy(acc.at[pl.ds(r0, rpt)], out_hbm.at[cid, pl.ds(r0, rpt)])


_BLK = 1000
_GRID = N // _BLK


def _dot(a, b):
    return jnp.dot(a, b, preferred_element_type=jnp.float32)


def _tc_pre_body(degp_ref, x_ref, fW1_ref, fb1_ref, fW2_ref, fb2_ref,
                 dinv_ref, g1_ref, x1_ref, y2_ref):
    deg = degp_ref[0] + degp_ref[1] + 1.0
    dinv_b = lax.rsqrt(deg)
    dinv_ref[...] = dinv_b
    x = x_ref[...]
    g1_ref[...] = x * dinv_b
    x1 = jnp.maximum(_dot(x, fW1_ref[...]) + fb1_ref[...], 0.0)
    x1_ref[...] = x1
    y2_ref[...] = _dot(x1, fW2_ref[...]) + fb2_ref[...]


def _tc_pre(degp, x, fW1, fb1, fW2, fb2):
    return pl.pallas_call(
        _tc_pre_body,
        grid=(_GRID,),
        in_specs=[
            pl.BlockSpec((NC, _BLK, IN_C), lambda i: (0, i, 0)),
            pl.BlockSpec((_BLK, IN_C), lambda i: (i, 0)),
            pl.BlockSpec((IN_C, HID_C), lambda i: (0, 0)),
            pl.BlockSpec((1, HID_C), lambda i: (0, 0)),
            pl.BlockSpec((HID_C, OUT_C), lambda i: (0, 0)),
            pl.BlockSpec((1, OUT_C), lambda i: (0, 0)),
        ],
        out_specs=[
            pl.BlockSpec((_BLK, IN_C), lambda i: (i, 0)),
            pl.BlockSpec((_BLK, IN_C), lambda i: (i, 0)),
            pl.BlockSpec((_BLK, HID_C), lambda i: (i, 0)),
            pl.BlockSpec((_BLK, OUT_C), lambda i: (i, 0)),
        ],
        out_shape=[
            jax.ShapeDtypeStruct((N, IN_C), jnp.float32),
            jax.ShapeDtypeStruct((N, IN_C), jnp.float32),
            jax.ShapeDtypeStruct((N, HID_C), jnp.float32),
            jax.ShapeDtypeStruct((N, OUT_C), jnp.float32),
        ],
    )(degp, x, fW1, fb1, fW2, fb2)


def _tc_mid_body(s1p_ref, g1_ref, dinv_ref, x1_ref, W1_ref, b1_ref, W2_ref,
                 beta1_ref, g2_ref):
    dinv_b = dinv_ref[...]
    s1 = (s1p_ref[0] + s1p_ref[1] + g1_ref[...]) * dinv_b
    h = jnp.maximum(_dot(s1, W1_ref[...]) + b1_ref[...], 0.0)
    beta1 = beta1_ref[0, 0]
    h = beta1 * h + (1.0 - beta1) * x1_ref[...]
    g2_ref[...] = _dot(h, W2_ref[...]) * dinv_b


def _tc_mid(s1p, g1, dinv_b, x1, W1, b1, W2, beta1):
    return pl.pallas_call(
        _tc_mid_body,
        grid=(_GRID,),
        in_specs=[
            pl.BlockSpec((NC, _BLK, IN_C), lambda i: (0, i, 0)),
            pl.BlockSpec((_BLK, IN_C), lambda i: (i, 0)),
            pl.BlockSpec((_BLK, IN_C), lambda i: (i, 0)),
            pl.BlockSpec((_BLK, HID_C), lambda i: (i, 0)),
            pl.BlockSpec((IN_C, HID_C), lambda i: (0, 0)),
            pl.BlockSpec((1, HID_C), lambda i: (0, 0)),
            pl.BlockSpec((HID_C, OUT_C), lambda i: (0, 0)),
            pl.BlockSpec((1, 1), lambda i: (0, 0)),
        ],
        out_specs=pl.BlockSpec((_BLK, OUT_C), lambda i: (i, 0)),
        out_shape=jax.ShapeDtypeStruct((N, OUT_C), jnp.float32),
    )(s1p, g1, dinv_b, x1, W1, b1, W2, beta1)


def _tc_post_body(s2p_ref, g2_ref, dinv_ref, y2_ref, b2_ref, beta2_ref,
                  out_ref):
    dinv_b = dinv_ref[...]
    h2 = (s2p_ref[0] + s2p_ref[1] + g2_ref[...]) * dinv_b + b2_ref[...]
    beta2 = beta2_ref[0, 0]
    o = beta2 * h2 + (1.0 - beta2) * y2_ref[...]
    m = jnp.max(o, axis=1, keepdims=True)
    z = o - m
    lse = jnp.log(jnp.sum(jnp.exp(z), axis=1, keepdims=True))
    out_ref[...] = z - lse


def _tc_post(s2p, g2, dinv_b, y2, b2, beta2):
    return pl.pallas_call(
        _tc_post_body,
        grid=(_GRID,),
        in_specs=[
            pl.BlockSpec((NC, _BLK, OUT_C), lambda i: (0, i, 0)),
            pl.BlockSpec((_BLK, OUT_C), lambda i: (i, 0)),
            pl.BlockSpec((_BLK, IN_C), lambda i: (i, 0)),
            pl.BlockSpec((_BLK, OUT_C), lambda i: (i, 0)),
            pl.BlockSpec((1, OUT_C), lambda i: (0, 0)),
            pl.BlockSpec((1, 1), lambda i: (0, 0)),
        ],
        out_specs=pl.BlockSpec((_BLK, OUT_C), lambda i: (i, 0)),
        out_shape=jax.ShapeDtypeStruct((N, OUT_C), jnp.float32),
    )(s2p, g2, dinv_b, y2, b2, beta2)


def kernel(x, W1, b1, W2, b2, fW1, fb1, fW2, fb2, beta1, beta2, edge_index):
    src = edge_index[0]
    dst = edge_index[1]
    npad = E_PAD - E
    pad_i = jnp.arange(npad, dtype=jnp.int32)
    src_p = jnp.concatenate([src, pad_i % N]).reshape(NW, STEPS, 1, WIN)
    dst_p = jnp.concatenate([dst, N + pad_i % (N_PAD - N)]).reshape(
        NW, STEPS, 1, WIN)
    ed = jnp.concatenate([src_p, dst_p], axis=2)
    dst_w = dst_p.reshape(NW, STEPS, WIN)

    zeros_row = jnp.zeros((N_PAD, IN_C), jnp.float32)

    degp = _make_sc_degree()(dst_w, zeros_row,
                             jnp.ones((WIN, IN_C), jnp.float32))
    dinv_b, g1, x1, y2 = _tc_pre(degp, x, fW1, fb1.reshape(1, HID_C),
                                 fW2, fb2.reshape(1, OUT_C))
    s1p = _make_sc_aggregate()(ed, g1, zeros_row)
    g2 = _tc_mid(s1p, g1, dinv_b, x1, W1, b1.reshape(1, HID_C),
                 W2, jnp.reshape(beta1, (1, 1)))
    s2p = _make_sc_aggregate()(ed, g2, zeros_row)
    out = _tc_post(s2p, g2, dinv_b, y2, b2.reshape(1, OUT_C),
                   jnp.reshape(beta2, (1, 1)))
    return out


# speedup vs baseline: 34.0582x; 1.1239x over previous
import functools

import jax
import jax.numpy as jnp
from jax import lax
from jax.experimental import pallas as pl
from jax.experimental.pallas import tpu as pltpu
from jax.experimental.pallas import tpu_sc as plsc

N = 10000
IN_C = 128
HID_C = 256
OUT_C = 128
E = 320000

NC = 2
NS = 16
NW = NC * NS
WIN = 128
STEPS = 80
EPT = WIN * STEPS
E_PAD = EPT * NW
N_PAD = 10240
DEG_W = 16
NBUF = 2


DEG_PACK = N_PAD * DEG_W // IN_C


@functools.cache
def _make_sc_degree():
    mesh = plsc.VectorSubcoreMesh(
        core_axis_name="core", subcore_axis_name="subcore")
    return pl.kernel(
        _sc_degree_body,
        out_type=jax.ShapeDtypeStruct((NC, DEG_PACK, IN_C), jnp.float32),
        mesh=mesh,
        scratch_types=[
            pltpu.VMEM_SHARED((N_PAD, DEG_W), jnp.float32),
            pltpu.VMEM((STEPS, WIN), jnp.int32),
            pltpu.VMEM((WIN, DEG_W), jnp.float32),
            pltpu.VMEM((N_PAD // NS, DEG_W), jnp.float32),
            pltpu.VMEM((DEG_PACK // NS, IN_C), jnp.float32),
        ],
    )


def _sc_degree_body(dst_hbm, out_hbm, acc, didx, ones, rbuf, stag):
    cid = lax.axis_index("core")
    sid = lax.axis_index("subcore")
    wid = cid * NS + sid
    rpt = N_PAD // NS
    r0 = sid * rpt
    pack = DEG_W // 16
    per = IN_C // DEG_W

    one_v = jnp.full((16,), 1.0, jnp.float32)
    zero_v = jnp.zeros((16,), jnp.float32)

    @pl.loop(0, WIN)
    def _(i):
        for c in range(pack):
            ones[i, pl.ds(c * 16, 16)] = one_v

    @pl.loop(0, rpt)
    def _(i):
        for c in range(pack):
            rbuf[i, pl.ds(c * 16, 16)] = zero_v

    pltpu.sync_copy(rbuf, acc.at[pl.ds(r0, rpt)])
    pltpu.sync_copy(dst_hbm.at[wid], didx)
    plsc.subcore_barrier()

    @pl.loop(0, STEPS)
    def _(t):
        pltpu.sync_copy(ones, acc.at[didx.at[t]], add=True)

    plsc.subcore_barrier()
    pltpu.sync_copy(acc.at[pl.ds(r0, rpt)], rbuf)

    @pl.loop(0, rpt // per)
    def _(r):
        for c in range(per):
            for k in range(pack):
                stag[r, pl.ds(c * DEG_W + k * 16, 16)] = (
                    rbuf[r * per + c, pl.ds(k * 16, 16)])

    pltpu.sync_copy(stag, out_hbm.at[cid, pl.ds(sid * (DEG_PACK // NS),
                                                DEG_PACK // NS)])


NSLOT = 4


@functools.cache
def _make_sc_aggregate():
    mesh = plsc.VectorSubcoreMesh(
        core_axis_name="core", subcore_axis_name="subcore")
    return pl.kernel(
        _sc_aggregate_body,
        out_type=jax.ShapeDtypeStruct((NC, N_PAD, IN_C), jnp.float32),
        mesh=mesh,
        scratch_types=[
            pltpu.VMEM_SHARED((N_PAD, IN_C), jnp.float32),
            pltpu.VMEM((NSLOT, 2, WIN), jnp.int32),
            pltpu.VMEM((NBUF, WIN, IN_C), jnp.float32),
            pltpu.SemaphoreType.DMA((NSLOT,)),
            pltpu.SemaphoreType.DMA((NBUF,)),
        ],
    )


def _sc_aggregate_body(ed_hbm, g_hbm, z_hbm, out_hbm,
                       acc, ebuf, rows, isem, gsem):
    cid = lax.axis_index("core")
    sid = lax.axis_index("subcore")
    wid = cid * NS + sid
    rpt = N_PAD // NS
    r0 = sid * rpt
    pltpu.sync_copy(z_hbm.at[pl.ds(r0, rpt)], acc.at[pl.ds(r0, rpt)])
    for k in range(NSLOT):
        pltpu.async_copy(ed_hbm.at[wid, k], ebuf.at[k], isem.at[k])
    for b in range(NBUF):
        pltpu.make_async_copy(ed_hbm.at[wid, b], ebuf.at[b], isem.at[b]).wait()
        pltpu.async_copy(g_hbm.at[ebuf.at[b, 0]], rows.at[b], gsem.at[b])
    plsc.subcore_barrier()

    @pl.loop(0, STEPS, step=NBUF)
    def _(t0):
        for b in range(NBUF):
            t = t0 + b
            slot = lax.rem(t, NSLOT)
            pltpu.make_async_copy(
                g_hbm.at[ebuf.at[slot, 0]], rows.at[b], gsem.at[b]).wait()
            pltpu.sync_copy(rows.at[b], acc.at[ebuf.at[slot, 1]], add=True)
            nt_idx = t + NSLOT

            @pl.when(nt_idx < STEPS)
            def _():
                pltpu.async_copy(
                    ed_hbm.at[wid, nt_idx], ebuf.at[slot], isem.at[slot])

            nt_g = t + NBUF

            @pl.when(nt_g < STEPS)
            def _():
                slot2 = lax.rem(nt_g, NSLOT)
                pltpu.make_async_copy(
                    ed_hbm.at[wid, nt_g], ebuf.at[slot2],
                    isem.at[slot2]).wait()
                pltpu.async_copy(
                    g_hbm.at[ebuf.at[slot2, 0]], rows.at[b], gsem.at[b])

    plsc.subcore_barrier()
    pltpu.sync_copy(acc.at[pl.ds(r0, rpt)], out_hbm.at[cid, pl.ds(r0, rpt)])


_BLK = 1000
_GRID = N // _BLK


def _dot(a, b):
    return jnp.dot(a, b, preferred_element_type=jnp.float32)


def _tc_pre_body(degp_ref, x_ref, fW1_ref, fb1_ref, fW2_ref, fb2_ref,
                 dinv_ref, g1_ref, x1_ref, y2_ref):
    deg = degp_ref[0, :, 0:1] + degp_ref[1, :, 0:1] + 1.0
    dinv_b = jnp.broadcast_to(lax.rsqrt(deg), (_BLK, IN_C))
    dinv_ref[...] = dinv_b
    x = x_ref[...]
    g1_ref[...] = x * dinv_b
    x1 = jnp.maximum(_dot(x, fW1_ref[...]) + fb1_ref[...], 0.0)
    x1_ref[...] = x1
    y2_ref[...] = _dot(x1, fW2_ref[...]) + fb2_ref[...]


def _tc_pre(degp, x, fW1, fb1, fW2, fb2):
    return pl.pallas_call(
        _tc_pre_body,
        grid=(_GRID,),
        in_specs=[
            pl.BlockSpec((NC, _BLK, DEG_W), lambda i: (0, i, 0)),
            pl.BlockSpec((_BLK, IN_C), lambda i: (i, 0)),
            pl.BlockSpec((IN_C, HID_C), lambda i: (0, 0)),
            pl.BlockSpec((1, HID_C), lambda i: (0, 0)),
            pl.BlockSpec((HID_C, OUT_C), lambda i: (0, 0)),
            pl.BlockSpec((1, OUT_C), lambda i: (0, 0)),
        ],
        out_specs=[
            pl.BlockSpec((_BLK, IN_C), lambda i: (i, 0)),
            pl.BlockSpec((_BLK, IN_C), lambda i: (i, 0)),
            pl.BlockSpec((_BLK, HID_C), lambda i: (i, 0)),
            pl.BlockSpec((_BLK, OUT_C), lambda i: (i, 0)),
        ],
        out_shape=[
            jax.ShapeDtypeStruct((N, IN_C), jnp.float32),
            jax.ShapeDtypeStruct((N, IN_C), jnp.float32),
            jax.ShapeDtypeStruct((N, HID_C), jnp.float32),
            jax.ShapeDtypeStruct((N, OUT_C), jnp.float32),
        ],
    )(degp, x, fW1, fb1, fW2, fb2)


def _tc_mid_body(s1p_ref, g1_ref, dinv_ref, x1_ref, W1_ref, b1_ref, W2_ref,
                 beta1_ref, g2_ref):
    dinv_b = dinv_ref[...]
    s1 = (s1p_ref[0] + s1p_ref[1] + g1_ref[...]) * dinv_b
    h = jnp.maximum(_dot(s1, W1_ref[...]) + b1_ref[...], 0.0)
    beta1 = beta1_ref[0, 0]
    h = beta1 * h + (1.0 - beta1) * x1_ref[...]
    g2_ref[...] = _dot(h, W2_ref[...]) * dinv_b


def _tc_mid(s1p, g1, dinv_b, x1, W1, b1, W2, beta1):
    return pl.pallas_call(
        _tc_mid_body,
        grid=(_GRID,),
        in_specs=[
            pl.BlockSpec((NC, _BLK, IN_C), lambda i: (0, i, 0)),
            pl.BlockSpec((_BLK, IN_C), lambda i: (i, 0)),
            pl.BlockSpec((_BLK, IN_C), lambda i: (i, 0)),
            pl.BlockSpec((_BLK, HID_C), lambda i: (i, 0)),
            pl.BlockSpec((IN_C, HID_C), lambda i: (0, 0)),
            pl.BlockSpec((1, HID_C), lambda i: (0, 0)),
            pl.BlockSpec((HID_C, OUT_C), lambda i: (0, 0)),
            pl.BlockSpec((1, 1), lambda i: (0, 0)),
        ],
        out_specs=pl.BlockSpec((_BLK, OUT_C), lambda i: (i, 0)),
        out_shape=jax.ShapeDtypeStruct((N, OUT_C), jnp.float32),
    )(s1p, g1, dinv_b, x1, W1, b1, W2, beta1)


def _tc_post_body(s2p_ref, g2_ref, dinv_ref, y2_ref, b2_ref, beta2_ref,
                  out_ref):
    dinv_b = dinv_ref[...]
    h2 = (s2p_ref[0] + s2p_ref[1] + g2_ref[...]) * dinv_b + b2_ref[...]
    beta2 = beta2_ref[0, 0]
    o = beta2 * h2 + (1.0 - beta2) * y2_ref[...]
    m = jnp.max(o, axis=1, keepdims=True)
    z = o - m
    lse = jnp.log(jnp.sum(jnp.exp(z), axis=1, keepdims=True))
    out_ref[...] = z - lse


def _tc_post(s2p, g2, dinv_b, y2, b2, beta2):
    return pl.pallas_call(
        _tc_post_body,
        grid=(_GRID,),
        in_specs=[
            pl.BlockSpec((NC, _BLK, OUT_C), lambda i: (0, i, 0)),
            pl.BlockSpec((_BLK, OUT_C), lambda i: (i, 0)),
            pl.BlockSpec((_BLK, IN_C), lambda i: (i, 0)),
            pl.BlockSpec((_BLK, OUT_C), lambda i: (i, 0)),
            pl.BlockSpec((1, OUT_C), lambda i: (0, 0)),
            pl.BlockSpec((1, 1), lambda i: (0, 0)),
        ],
        out_specs=pl.BlockSpec((_BLK, OUT_C), lambda i: (i, 0)),
        out_shape=jax.ShapeDtypeStruct((N, OUT_C), jnp.float32),
    )(s2p, g2, dinv_b, y2, b2, beta2)


def kernel(x, W1, b1, W2, b2, fW1, fb1, fW2, fb2, beta1, beta2, edge_index):
    src = edge_index[0]
    dst = edge_index[1]
    npad = E_PAD - E
    pad_i = jnp.arange(npad, dtype=jnp.int32)
    src_p = jnp.concatenate([src, pad_i % N]).reshape(NW, STEPS, 1, WIN)
    dst_p = jnp.concatenate([dst, N + pad_i % (N_PAD - N)]).reshape(
        NW, STEPS, 1, WIN)
    ed = jnp.concatenate([src_p, dst_p], axis=2)
    dst_w = dst_p.reshape(NW, STEPS, WIN)

    zeros_row = jnp.zeros((N_PAD, IN_C), jnp.float32)

    degp = _make_sc_degree()(dst_w).reshape(NC, N_PAD, DEG_W)
    dinv_b, g1, x1, y2 = _tc_pre(degp, x, fW1, fb1.reshape(1, HID_C),
                                 fW2, fb2.reshape(1, OUT_C))
    s1p = _make_sc_aggregate()(ed, g1, zeros_row)
    g2 = _tc_mid(s1p, g1, dinv_b, x1, W1, b1.reshape(1, HID_C),
                 W2, jnp.reshape(beta1, (1, 1)))
    s2p = _make_sc_aggregate()(ed, g2, zeros_row)
    out = _tc_post(s2p, g2, dinv_b, y2, b2.reshape(1, OUT_C),
                   jnp.reshape(beta2, (1, 1)))
    return out
